# baseline (speedup 1.0000x reference)
import functools

import jax
import jax.numpy as jnp
from jax import lax
from jax.experimental import pallas as pl
from jax.experimental.pallas import tpu as pltpu
from jax.experimental.pallas import tpu_sc as plsc

_N = 4096
_ROWS = 128
_GRID = _N // _ROWS
_NC, _NS, _L = 2, 16, 16
_NW = _NC * _NS
_RPW = _N // _NW


def _mine_body(x_ref, x1_ref, a_p_ref, a_n_ref, g_p_ref, g_n_ref):
    i = pl.program_id(0)
    x = x_ref[...]
    cols = lax.broadcasted_iota(jnp.int32, x.shape, 1)
    rows = lax.broadcasted_iota(jnp.int32, x.shape, 0) + i * _ROWS
    diag = cols == rows
    colsf = cols.astype(jnp.float32)
    mod_p = x + jnp.where(diag, 0.0, 9999999.0)
    mod_n = x + jnp.where(diag, -9999999.0, 0.0)
    hard_p = jnp.min(mod_p, axis=1)
    hard_n = jnp.max(mod_n, axis=1)
    bigf = jnp.float32(3e38)
    apf = jnp.min(jnp.where(mod_p == hard_p[:, None], colsf, bigf), axis=1)
    anf = jnp.min(jnp.where(mod_n == hard_n[:, None], colsf, bigf), axis=1)
    x1 = x1_ref[...]
    zero = jnp.float32(0.0)
    onep = jnp.where(colsf == apf[:, None], x1, zero)
    onen = jnp.where(colsf == anf[:, None], x1, zero)
    ones = jnp.ones((_N,), jnp.float32)
    dn = (((1,), (0,)), ((), ()))
    g_p_ref[...] = lax.dot_general(onep, ones, dn, preferred_element_type=jnp.float32)
    g_n_ref[...] = lax.dot_general(onen, ones, dn, preferred_element_type=jnp.float32)
    m = jnp.maximum(hard_p, hard_n)
    lse = m + jnp.log(jnp.exp(hard_p - m) + jnp.exp(hard_n - m))
    scale = jnp.float32(-1.0 / _N)
    a_p_ref[...] = (hard_p - lse) * scale
    a_n_ref[...] = (hard_n - lse) * scale


def _sc_body(a_p, a_n, g_p, g_n, out, ap_v, an_v, gp_v, gn_v, acc_v):
    wid = lax.axis_index("s") * _NC + lax.axis_index("c")
    base = wid * _RPW
    pltpu.sync_copy(a_p.at[pl.ds(base, _RPW)], ap_v)
    pltpu.sync_copy(a_n.at[pl.ds(base, _RPW)], an_v)
    pltpu.sync_copy(g_p.at[pl.ds(base, _RPW)], gp_v)
    pltpu.sync_copy(g_n.at[pl.ds(base, _RPW)], gn_v)
    acc = jnp.zeros((_L,), jnp.float32)
    for k in range(_RPW // _L):
        s = pl.ds(k * _L, _L)
        smp = 1.0 / (1.0 + jnp.exp(gn_v[s] - gp_v[s]))
        smn = 1.0 - smp
        acc = acc + smp * ap_v[s] + smn * an_v[s]
    acc_v[...] = acc
    pltpu.sync_copy(acc_v, out.at[wid])


@functools.cache
def _get_sc_gather():
    return functools.partial(
        pl.kernel,
        out_type=jax.ShapeDtypeStruct((_NW, _L), jnp.float32),
        mesh=plsc.VectorSubcoreMesh(
            core_axis_name="c", subcore_axis_name="s",
            num_cores=_NC, num_subcores=_NS),
        scratch_types=[
            pltpu.VMEM((_RPW,), jnp.float32),
            pltpu.VMEM((_RPW,), jnp.float32),
            pltpu.VMEM((_RPW,), jnp.float32),
            pltpu.VMEM((_RPW,), jnp.float32),
            pltpu.VMEM((_L,), jnp.float32),
        ],
    )(_sc_body)


def _sum_body(p_ref, o_ref):
    o_ref[0, 0] = jnp.sum(p_ref[...])


def kernel(sim_matrix0, sim_matrix1):
    a_p, a_n, g_p, g_n = pl.pallas_call(
        _mine_body,
        grid=(_GRID,),
        in_specs=[pl.BlockSpec((_ROWS, _N), lambda i: (i, 0))] * 2,
        out_specs=[pl.BlockSpec((_ROWS,), lambda i: (i,))] * 4,
        out_shape=[
            jax.ShapeDtypeStruct((_N,), jnp.float32),
            jax.ShapeDtypeStruct((_N,), jnp.float32),
            jax.ShapeDtypeStruct((_N,), jnp.float32),
            jax.ShapeDtypeStruct((_N,), jnp.float32),
        ],
    )(sim_matrix0, sim_matrix1)
    partials = _get_sc_gather()(a_p, a_n, g_p, g_n)
    loss = pl.pallas_call(
        _sum_body,
        out_specs=pl.BlockSpec(memory_space=pltpu.SMEM),
        out_shape=jax.ShapeDtypeStruct((1, 1), jnp.float32),
    )(partials)
    return loss.reshape(())


# speedup vs baseline: 1.0935x; 1.0935x over previous
import functools

import jax
import jax.numpy as jnp
from jax import lax
from jax.experimental import pallas as pl
from jax.experimental.pallas import tpu as pltpu
from jax.experimental.pallas import tpu_sc as plsc

_N = 4096
_ROWS = 256
_GRID = _N // _ROWS
_NC, _NS, _L = 2, 16, 16
_NW = _NC * _NS
_RPW = _N // _NW


def _mine_body(x_ref, x1_ref, a_p_ref, a_n_ref, g_p_ref, g_n_ref):
    i = pl.program_id(0)
    x = x_ref[...]
    cols = lax.broadcasted_iota(jnp.int32, x.shape, 1)
    rows = lax.broadcasted_iota(jnp.int32, x.shape, 0) + i * _ROWS
    diag = cols == rows
    colsf = cols.astype(jnp.float32)
    mod_p = x + jnp.where(diag, 0.0, 9999999.0)
    mod_n = x + jnp.where(diag, -9999999.0, 0.0)
    hard_p = jnp.min(mod_p, axis=1)
    hard_n = jnp.max(mod_n, axis=1)
    bigf = jnp.float32(3e38)
    apf = jnp.min(jnp.where(mod_p == hard_p[:, None], colsf, bigf), axis=1)
    anf = jnp.min(jnp.where(mod_n == hard_n[:, None], colsf, bigf), axis=1)
    x1 = x1_ref[...]
    zero = jnp.float32(0.0)
    onep = jnp.where(colsf == apf[:, None], x1, zero)
    onen = jnp.where(colsf == anf[:, None], x1, zero)
    ones = jnp.ones((_N,), jnp.float32)
    dn = (((1,), (0,)), ((), ()))
    g_p_ref[...] = lax.dot_general(onep, ones, dn, preferred_element_type=jnp.float32)
    g_n_ref[...] = lax.dot_general(onen, ones, dn, preferred_element_type=jnp.float32)
    m = jnp.maximum(hard_p, hard_n)
    lse = m + jnp.log(jnp.exp(hard_p - m) + jnp.exp(hard_n - m))
    scale = jnp.float32(-1.0 / _N)
    a_p_ref[...] = (hard_p - lse) * scale
    a_n_ref[...] = (hard_n - lse) * scale


def _sc_body(a_p, a_n, g_p, g_n, out, ap_v, an_v, gp_v, gn_v, acc_v):
    wid = lax.axis_index("s") * _NC + lax.axis_index("c")
    base = wid * _RPW
    pltpu.sync_copy(a_p.at[pl.ds(base, _RPW)], ap_v)
    pltpu.sync_copy(a_n.at[pl.ds(base, _RPW)], an_v)
    pltpu.sync_copy(g_p.at[pl.ds(base, _RPW)], gp_v)
    pltpu.sync_copy(g_n.at[pl.ds(base, _RPW)], gn_v)
    acc = jnp.zeros((_L,), jnp.float32)
    for k in range(_RPW // _L):
        s = pl.ds(k * _L, _L)
        smp = 1.0 / (1.0 + jnp.exp(gn_v[s] - gp_v[s]))
        smn = 1.0 - smp
        acc = acc + smp * ap_v[s] + smn * an_v[s]
    acc_v[...] = acc
    pltpu.sync_copy(acc_v, out.at[wid])


@functools.cache
def _get_sc_gather():
    return functools.partial(
        pl.kernel,
        out_type=jax.ShapeDtypeStruct((_NW, _L), jnp.float32),
        mesh=plsc.VectorSubcoreMesh(
            core_axis_name="c", subcore_axis_name="s",
            num_cores=_NC, num_subcores=_NS),
        scratch_types=[
            pltpu.VMEM((_RPW,), jnp.float32),
            pltpu.VMEM((_RPW,), jnp.float32),
            pltpu.VMEM((_RPW,), jnp.float32),
            pltpu.VMEM((_RPW,), jnp.float32),
            pltpu.VMEM((_L,), jnp.float32),
        ],
    )(_sc_body)


def _sum_body(p_ref, o_ref):
    o_ref[0, 0] = jnp.sum(p_ref[...])


def kernel(sim_matrix0, sim_matrix1):
    a_p, a_n, g_p, g_n = pl.pallas_call(
        _mine_body,
        grid=(_GRID,),
        in_specs=[pl.BlockSpec((_ROWS, _N), lambda i: (i, 0))] * 2,
        out_specs=[pl.BlockSpec((_ROWS,), lambda i: (i,))] * 4,
        out_shape=[
            jax.ShapeDtypeStruct((_N,), jnp.float32),
            jax.ShapeDtypeStruct((_N,), jnp.float32),
            jax.ShapeDtypeStruct((_N,), jnp.float32),
            jax.ShapeDtypeStruct((_N,), jnp.float32),
        ],
    )(sim_matrix0, sim_matrix1)
    partials = _get_sc_gather()(a_p, a_n, g_p, g_n)
    loss = pl.pallas_call(
        _sum_body,
        out_specs=pl.BlockSpec(memory_space=pltpu.SMEM),
        out_shape=jax.ShapeDtypeStruct((1, 1), jnp.float32),
    )(partials)
    return loss.reshape(())
